# Initial kernel scaffold; baseline (speedup 1.0000x reference)
import jax
import jax.numpy as jnp
from jax.experimental import pallas as pl


def kernel(x, edge_index, params):
    raise NotImplementedError("write your pallas kernel here")


# speedup vs baseline: 2.7498x; 2.7498x over previous
import functools

import jax
import jax.numpy as jnp
from jax import lax
from jax.experimental import pallas as pl
from jax.experimental.pallas import tpu as pltpu
from jax.experimental.pallas import tpu_sc as plsc

N = 10000
E = 320000
E2 = E + N
D = 128
MP_F = [32, 32, 32, 32, 16, 16, 16, 16, 8, 8, 8, 8]
DELTA = 2.0

NC = 2
NS = 16
NW = NC * NS
NP = 10240
RPS = NP // NS
CH = 128
RC = 16

FP_BASE = 10000
LP_BASE = 10016
TR_BASE = 10032

_B_COMMON = [20736 * k for k in range(13)]
BOUNDS32 = _B_COMMON + [269184, 289536, 309888, E2]
BOUNDS16 = _B_COMMON + [269136, 289440, 309744, E2]

E2P = 330240
WPW = E2P // NW
WCH = 120
WNCH = WPW // WCH


def _mesh():
  return plsc.VectorSubcoreMesh(core_axis_name="c", subcore_axis_name="s",
                                num_cores=NC, num_subcores=NS)


def _bcast(vec, lane):
  dn = lax.GatherDimensionNumbers(offset_dims=(), collapsed_slice_dims=(0,),
                                  start_index_map=(0,))
  return lax.gather(vec, lane[:, None], dn, (1,),
                    mode=lax.GatherScatterMode.PROMISE_IN_BOUNDS)


def _sel(s, vals):
  r = jnp.int32(vals[0])
  for k in range(1, len(vals)):
    r = jnp.where(s == k, jnp.int32(vals[k]), r)
  return r


def _sc_deg_call(col):
  onehot = jnp.zeros((80, 8), jnp.float32).at[:, 0].set(1.0)
  zeros = jnp.zeros((RPS, 8), jnp.float32)
  epw = E // NW
  nch = epw // 80

  @functools.partial(
      pl.kernel,
      mesh=_mesh(),
      out_type=jax.ShapeDtypeStruct((NC, NP, 8), jnp.float32),
      compiler_params=pltpu.CompilerParams(use_tc_tiling_on_sc=False),
      scratch_types=[
          pltpu.VMEM((1, 80), jnp.int32),
          pltpu.VMEM((80, 8), jnp.float32),
          pltpu.VMEM_SHARED((NP, 8), jnp.float32),
      ],
  )
  def k(col_hbm, onehot_hbm, zeros_hbm, out_hbm, col_v, rows_v, agg_sh):
    c = lax.axis_index("c")
    s = lax.axis_index("s")
    base = (s * NC + c) * epw
    pltpu.sync_copy(zeros_hbm, agg_sh.at[pl.ds(s * RPS, RPS)])
    pltpu.sync_copy(onehot_hbm, rows_v)
    plsc.subcore_barrier()

    def body(j, carry):
      pltpu.sync_copy(col_hbm.at[pl.ds(pl.multiple_of(base + j * 80, 8), 80)],
                      col_v.at[0])
      pltpu.sync_copy(rows_v, agg_sh.at[col_v.at[0]], add=True)
      return carry

    lax.fori_loop(0, nch, body, 0)
    plsc.subcore_barrier()
    pltpu.sync_copy(agg_sh.at[pl.ds(s * RPS, RPS)],
                    out_hbm.at[c, pl.ds(s * RPS, RPS)])

  return k(col, onehot, zeros)


def _sc_wsort_call(dinv8ext, srow_g, scol_g):
  @functools.partial(
      pl.kernel,
      mesh=_mesh(),
      out_type=jax.ShapeDtypeStruct((E2P, 8), jnp.float32),
      compiler_params=pltpu.CompilerParams(use_tc_tiling_on_sc=False),
      scratch_types=[
          pltpu.VMEM((WCH,), jnp.int32),
          pltpu.VMEM((WCH,), jnp.int32),
          pltpu.VMEM((WCH, 8), jnp.float32),
          pltpu.VMEM((WCH, 8), jnp.float32),
          pltpu.SemaphoreType.DMA,
          pltpu.SemaphoreType.DMA,
      ],
  )
  def k(tab_hbm, sr_hbm, sc_hbm, out_hbm, ri_v, ci_v, dr_v, dc_v, sem1, sem2):
    c = lax.axis_index("c")
    s = lax.axis_index("s")
    base = (s * NC + c) * WPW

    def body(j, carry):
      pos = pl.multiple_of(base + j * WCH, 8)
      pltpu.sync_copy(sr_hbm.at[pl.ds(pos, WCH)], ri_v)
      pltpu.sync_copy(sc_hbm.at[pl.ds(pos, WCH)], ci_v)
      cp1 = pltpu.async_copy(tab_hbm.at[ri_v], dr_v, sem1)
      cp2 = pltpu.async_copy(tab_hbm.at[ci_v], dc_v, sem2)
      cp1.wait()
      cp2.wait()

      def vops(t, carry2):
        u = dr_v[pl.ds(t * 16, 16)] * dc_v[pl.ds(t * 16, 16)]
        dr_v[pl.ds(t * 16, 16)] = u * jnp.float32(-DELTA)
        return carry2

      lax.fori_loop(0, WCH * 8 // 16, vops, 0)
      pltpu.sync_copy(dr_v, out_hbm.at[pl.ds(pos, WCH)])
      return carry

    lax.fori_loop(0, WNCH, body, 0)

  return k(dinv8ext, srow_g, scol_g)


def _sc_layer_call(hl, srow, sidx, same, wflat, F, bounds):
  zeros = jnp.zeros((RPS, F), jnp.float32)
  starts = bounds[:16]
  sizes = [bounds[k + 1] - bounds[k] for k in range(16)]
  nfulls = [sz // CH for sz in sizes]
  nrems = [(sz % CH) // RC for sz in sizes]
  nacc = F // 16

  @functools.partial(
      pl.kernel,
      mesh=_mesh(),
      out_type=jax.ShapeDtypeStruct((NP, F), jnp.float32),
      compiler_params=pltpu.CompilerParams(use_tc_tiling_on_sc=False),
      scratch_types=[
          pltpu.VMEM((CH,), jnp.int32),
          pltpu.VMEM((1, CH), jnp.int32),
          pltpu.VMEM((CH,), jnp.float32),
          pltpu.VMEM((CH,), jnp.float32),
          pltpu.VMEM((CH, F), jnp.float32),
          pltpu.VMEM((CH, F), jnp.float32),
          pltpu.VMEM((RC,), jnp.int32),
          pltpu.VMEM((1, RC), jnp.int32),
          pltpu.VMEM((RC,), jnp.float32),
          pltpu.VMEM((RC,), jnp.float32),
          pltpu.VMEM((RC, F), jnp.float32),
          pltpu.VMEM((RC, F), jnp.float32),
          pltpu.VMEM_SHARED((NP, F), jnp.float32),
          pltpu.SemaphoreType.DMA,
      ],
  )
  def k(hl_hbm, srow_hbm, sidx_hbm, same_hbm, w_hbm, zeros_hbm, out_hbm,
        ri_v, ti_v, w_v, s_v, rows_v, pre_v,
        ri_r, ti_r, w_r, s_r, rows_r, pre_r, agg_sh, sem):
    c = lax.axis_index("c")
    s = lax.axis_index("s")

    @pl.when(c == 0)
    def _():
      pltpu.sync_copy(zeros_hbm, agg_sh.at[pl.ds(s * RPS, RPS)])
      plsc.subcore_barrier()
      start = _sel(s, starts)
      nfull = _sel(s, nfulls)
      nrem = _sel(s, nrems)

      def scan_rows(nrows, rv, pv, wv, sv, acc):
        def group(jj, acc2):
          w16 = wv[pl.ds(jj * 16, 16)]
          s16 = sv[pl.ds(jj * 16, 16)]
          for j16 in range(16):
            rowi = jj * 16 + j16
            lane = jnp.full((16,), j16, jnp.int32)
            wj = _bcast(w16, lane)
            sj = _bcast(s16, lane)
            nxt = []
            for f in range(nacc):
              r = rv[rowi, pl.ds(f * 16, 16)]
              m = r * wj
              a = acc2[f] * sj + m
              pv[rowi, pl.ds(f * 16, 16)] = a
              nxt.append(a)
            acc2 = tuple(nxt)
          return acc2
        return lax.fori_loop(0, nrows // 16, group, acc)

      def chunk(pos, nrows, riv, tiv, wv, sv, rv, pv, acc):
        pos = pl.multiple_of(pos, 8)
        pltpu.sync_copy(srow_hbm.at[pl.ds(pos, nrows)], riv)
        pltpu.sync_copy(sidx_hbm.at[pl.ds(pos, nrows)], tiv.at[0])
        pltpu.sync_copy(w_hbm.at[pl.ds(pos, nrows)], wv)
        pltpu.sync_copy(same_hbm.at[pl.ds(pos, nrows)], sv)
        pltpu.async_copy(hl_hbm.at[riv], rv, sem).wait()
        acc = scan_rows(nrows, rv, pv, wv, sv, acc)
        pltpu.sync_copy(pv, agg_sh.at[tiv.at[0]], add=True)
        return acc

      acc0 = tuple(jnp.zeros((16,), jnp.float32) for _ in range(nacc))

      def full_body(j, acc):
        return chunk(start + j * CH, CH, ri_v, ti_v, w_v, s_v,
                     rows_v, pre_v, acc)

      acc1 = lax.fori_loop(0, nfull, full_body, acc0)

      def rem_body(j, acc):
        return chunk(start + nfull * CH + j * RC, RC, ri_r, ti_r, w_r,
                     s_r, rows_r, pre_r, acc)

      lax.fori_loop(0, nrem, rem_body, acc1)
      plsc.subcore_barrier()
      pltpu.sync_copy(agg_sh.at[pl.ds(s * RPS, RPS)],
                      out_hbm.at[pl.ds(s * RPS, RPS)])

  return k(hl, srow, sidx, same, wflat, zeros)


BN = 1000


def _tc0_call(x, W0, b0, Wc0, deg8):
  F = Wc0.shape[1]

  def body(x_r, w0_r, b0_r, wc_r, dg_r, hl_r, dinv8_r):
    h = jnp.dot(x_r[...], w0_r[...], preferred_element_type=jnp.float32)
    h = h + b0_r[...]
    hl = jnp.dot(h, wc_r[...], preferred_element_type=jnp.float32)
    deg = dg_r[0, :, 0:1] + dg_r[1, :, 0:1]
    dinv = jnp.where(deg > 0.0,
                     1.0 / jnp.sqrt(jnp.maximum(deg, 1e-12)), 0.0)
    hl_r[...] = hl
    dinv8_r[...] = jnp.broadcast_to(dinv, dinv8_r.shape)

  return pl.pallas_call(
      body,
      grid=(N // BN,),
      in_specs=[
          pl.BlockSpec((BN, D), lambda i: (i, 0)),
          pl.BlockSpec((D, D), lambda i: (0, 0)),
          pl.BlockSpec((1, D), lambda i: (0, 0)),
          pl.BlockSpec((D, F), lambda i: (0, 0)),
          pl.BlockSpec((NC, BN, 8), lambda i: (0, i, 0)),
      ],
      out_specs=[
          pl.BlockSpec((BN, F), lambda i: (i, 0)),
          pl.BlockSpec((BN, 8), lambda i: (i, 0)),
      ],
      out_shape=[
          jax.ShapeDtypeStruct((N, F), jnp.float32),
          jax.ShapeDtypeStruct((N, 8), jnp.float32),
      ],
  )(x, W0, b0.reshape(1, D), Wc0, deg8)


def _tc_layer_call(part, corr, bc, Wc):
  Fin = corr.shape[1]
  Fout = Wc.shape[1]

  def body(p_r, c_r, bc_r, wc_r, hln_r):
    agg = (p_r[...] + c_r[...]) + bc_r[...]
    h = jnp.tanh(agg)
    hln_r[...] = jnp.dot(h, wc_r[...], preferred_element_type=jnp.float32)

  return pl.pallas_call(
      body,
      grid=(N // BN,),
      in_specs=[
          pl.BlockSpec((BN, Fin), lambda i: (i, 0)),
          pl.BlockSpec((BN, Fin), lambda i: (i, 0)),
          pl.BlockSpec((1, Fin), lambda i: (0, 0)),
          pl.BlockSpec((Fin, Fout), lambda i: (0, 0)),
      ],
      out_specs=pl.BlockSpec((BN, Fout), lambda i: (i, 0)),
      out_shape=jax.ShapeDtypeStruct((N, Fout), jnp.float32),
  )(part[:N], corr, bc.reshape(1, Fin), Wc)


def _tc_extract_call(w8):
  BW = 2064

  def body(i_r, o_r):
    o_r[...] = i_r[:, 0:1]

  return pl.pallas_call(
      body,
      grid=(E2P // BW,),
      in_specs=[pl.BlockSpec((BW, 8), lambda i: (i, 0))],
      out_specs=pl.BlockSpec((BW, 1), lambda i: (i, 0)),
      out_shape=jax.ShapeDtypeStruct((E2P, 1), jnp.float32),
  )(w8)


def _tc_final_call(part, corr, bc, Wm0, bm0, Wm1, bm1, Wf, bf):
  Fin = corr.shape[1]

  def body(p_r, c_r, bc_r, wm0_r, bm0_r, wm1_r, bm1_r, wf_r, bf_r, out_r):
    agg = (p_r[...] + c_r[...]) + bc_r[...]
    h = jnp.tanh(agg)
    h = jnp.maximum(
        jnp.dot(h, wm0_r[...], preferred_element_type=jnp.float32)
        + bm0_r[...], 0.0)
    h = jnp.maximum(
        jnp.dot(h, wm1_r[...], preferred_element_type=jnp.float32)
        + bm1_r[...], 0.0)
    score = jnp.dot(h, wf_r[...], preferred_element_type=jnp.float32)
    out_r[...] = jnp.tanh(score + bf_r[...])

  u0 = Wm0.shape[1]
  u1 = Wm1.shape[1]
  return pl.pallas_call(
      body,
      grid=(N // BN,),
      in_specs=[
          pl.BlockSpec((BN, Fin), lambda i: (i, 0)),
          pl.BlockSpec((BN, Fin), lambda i: (i, 0)),
          pl.BlockSpec((1, Fin), lambda i: (0, 0)),
          pl.BlockSpec((Fin, u0), lambda i: (0, 0)),
          pl.BlockSpec((1, u0), lambda i: (0, 0)),
          pl.BlockSpec((u0, u1), lambda i: (0, 0)),
          pl.BlockSpec((1, u1), lambda i: (0, 0)),
          pl.BlockSpec((u1, 1), lambda i: (0, 0)),
          pl.BlockSpec((1, 1), lambda i: (0, 0)),
      ],
      out_specs=pl.BlockSpec((BN, 1), lambda i: (i, 0)),
      out_shape=jax.ShapeDtypeStruct((N, 1), jnp.float32),
  )(part[:N], corr, bc.reshape(1, Fin), Wm0, bm0.reshape(1, u0),
    Wm1, bm1.reshape(1, u1), Wf, bf.reshape(1, 1))


def _build_pos_arrays(scol, bounds):
  pos = jnp.arange(E2, dtype=jnp.int32)
  barr = jnp.asarray(bounds, jnp.int32)
  shard = jnp.searchsorted(barr[1:-1], pos, side='right').astype(jnp.int32)
  start = barr[shard]
  end = barr[shard + 1]
  prev_eq = jnp.concatenate(
      [jnp.zeros((1,), bool), scol[1:] == scol[:-1]])
  next_eq = jnp.concatenate(
      [scol[:-1] == scol[1:], jnp.zeros((1,), bool)])
  is_start = pos == start
  same01 = jnp.where(prev_eq & ~is_start, 1.0, 0.0).astype(jnp.float32)
  is_end = (~next_eq) | (pos + 1 == end)
  first_node = scol[start]
  last_node = scol[end - 1]
  in_first = scol == first_node
  in_last = (scol == last_node) & ~in_first
  tgt = jnp.where(in_first, FP_BASE + shard,
                  jnp.where(in_last, LP_BASE + shard, scol))
  sidx = jnp.where(is_end, tgt, TR_BASE + (pos % 16)).astype(jnp.int32)
  return sidx, same01


def _merge_corr(out, scol, bounds, F):
  corr = jnp.zeros((N, F), jnp.float32)
  for s in range(16):
    first_node = scol[bounds[s]]
    last_node = scol[bounds[s + 1] - 1]
    corr = corr.at[first_node].add(out[FP_BASE + s])
    corr = corr.at[last_node].add(out[LP_BASE + s])
  return corr


def kernel(x, edge_index, params):
  row = edge_index[0]
  col = edge_index[1]
  loop = jnp.arange(N, dtype=jnp.int32)
  row2 = jnp.concatenate([row, loop])
  col2 = jnp.concatenate([col, loop])
  perm = jnp.argsort(col2)
  scol = col2[perm]
  srow = row2[perm]
  selfloop = perm >= E
  srow_g = jnp.where(selfloop, N, srow)
  scol_g = jnp.where(selfloop, N + 1, scol)
  padi = jnp.zeros((E2P - E2,), jnp.int32)
  srow_gp = jnp.concatenate([srow_g, padi])
  scol_gp = jnp.concatenate([scol_g, padi])

  sidx32, same32 = _build_pos_arrays(scol, BOUNDS32)
  sidx16, same16 = _build_pos_arrays(scol, BOUNDS16)

  deg8 = _sc_deg_call(col)
  hl, dinv8 = _tc0_call(x, params['W0'], params['b0'], params['Wc0'], deg8)

  dinv8ext = jnp.concatenate([
      dinv8,
      jnp.ones((1, 8), jnp.float32),
      jnp.full((1, 8), -0.5, jnp.float32),
      jnp.zeros((6, 8), jnp.float32),
  ])
  w8 = _sc_wsort_call(dinv8ext, srow_gp, scol_gp)
  wflat = _tc_extract_call(w8).reshape(E2P)

  nl = len(MP_F)
  part = None
  corr = None
  for i in range(nl):
    freal = MP_F[i]
    fsc = max(freal, 16)
    bounds = BOUNDS32 if freal == 32 else BOUNDS16
    sidx, same = (sidx32, same32) if freal == 32 else (sidx16, same16)
    if fsc != hl.shape[1]:
      hl = jnp.pad(hl, ((0, 0), (0, fsc - hl.shape[1])))
    part = _sc_layer_call(hl, srow, sidx, same, wflat, fsc, bounds)
    corr = _merge_corr(part, scol, bounds, fsc)
    if i + 1 < nl:
      Wc = params['Wc%d' % (i + 1)]
      bc = params['bc%d' % i]
      if fsc != freal:
        Wc = jnp.pad(Wc, ((0, fsc - freal), (0, 0)))
        bc = jnp.pad(bc, (0, fsc - freal))
      if Wc.shape[1] < 16:
        Wc = jnp.pad(Wc, ((0, 0), (0, 16 - Wc.shape[1])))
      hl = _tc_layer_call(part, corr, bc, Wc)

  bc = jnp.pad(params['bc11'], (0, 8))
  Wm0 = jnp.pad(params['Wm0'], ((0, 8), (0, 0)))
  return _tc_final_call(part, corr, bc, Wm0, params['bm0'],
                        params['Wm1'], params['bm1'],
                        params['Wf'], params['bf'])
